# Initial kernel scaffold; baseline (speedup 1.0000x reference)
import jax
import jax.numpy as jnp
from jax.experimental import pallas as pl


def kernel(x, edge_index2, edge_attr2, batch, params):
    raise NotImplementedError("write your pallas kernel here")


# speedup vs baseline: 6.8974x; 6.8974x over previous
import functools

import jax
import jax.numpy as jnp
from jax import lax
from jax.experimental import pallas as pl
from jax.experimental.pallas import tpu as pltpu
from jax.experimental.pallas import tpu_sc as plsc

N = 10000
E = 320000
NINP = 128
NE = 4
NOUT1 = 24
NOUT2 = 24
NIN = NOUT1 + NOUT2
NUM_GRAPHS = 64

NW = 32
NS = 16
CHUNK = 128
EPW = E // NW
NCW = 80
AROWS = 2560
RPT = AROWS // NS

_F32 = jnp.float32
_BF16 = jnp.bfloat16
_HIGH = lax.Precision.HIGHEST


def _dotb(a, b):
    return jnp.dot(a.astype(_BF16), b.astype(_BF16),
                   preferred_element_type=_F32)


def _sc_edge_body(y_hbm, src_hbm, drow_hbm, dcol_hbm, ea_hbm, out_hbm,
                  src_v, dcol_v, drow_v, rows0, rows1, msg_v, accum,
                  sem0, sem1, sea0, sea1, ea0, ea1):
    cid = lax.axis_index("c")
    sid = lax.axis_index("s")
    wid = cid * NS + sid

    def _zmsg(r, _):
        for kk in range(8):
            msg_v[r, pl.ds(16 * kk, 16)] = jnp.zeros((16,), _F32)
        return _
    lax.fori_loop(0, CHUNK, _zmsg, None)
    pltpu.sync_copy(msg_v, accum.at[pl.ds(sid * RPT, CHUNK)])
    pltpu.sync_copy(msg_v.at[pl.ds(0, RPT - CHUNK)],
                    accum.at[pl.ds(sid * RPT + CHUNK, RPT - CHUNK)])

    ew = NCW * CHUNK
    pltpu.sync_copy(src_hbm.at[pl.ds(wid * ew, ew)], src_v)
    pltpu.sync_copy(dcol_hbm.at[pl.ds(wid * ew, ew)], dcol_v)
    pltpu.sync_copy(drow_hbm.at[pl.ds(wid * NCW, NCW)], drow_v)
    plsc.subcore_barrier()

    def _gather(j, rows_buf, sem, ea_buf, sea):
        h1 = pltpu.async_copy(y_hbm.at[src_v.at[pl.ds(j * CHUNK, CHUNK)]],
                              rows_buf, sem)
        h2 = pltpu.async_copy(
            ea_hbm.at[pl.ds((wid * NCW + j) * CHUNK * NE, CHUNK * NE)],
            ea_buf, sea)
        return h1, h2

    def _wait(handles):
        for h in handles:
            h.wait()

    def _combine_scatter(j, rows_buf, ea_buf):
        def _group(g, _):
            e0 = g * 16
            dcol_vec = dcol_v[pl.ds(j * CHUNK + e0, 16)]
            eav = [ea_buf[pl.ds(g * 64 + 16 * kk, 16)] for kk in range(4)]
            for q in range(16):
                p = 4 * q
                vq = eav[p // 16]
                a0 = vq[p % 16 + 0]
                a1 = vq[p % 16 + 1]
                a2 = vq[p % 16 + 2]
                a3 = vq[p % 16 + 3]
                col = dcol_vec[q]
                e = e0 + q
                acc = []
                for h in range(2):
                    o = 16 * h
                    acc.append(a0 * rows_buf[e, pl.ds(o, 16)]
                               + a1 * rows_buf[e, pl.ds(32 + o, 16)]
                               + a2 * rows_buf[e, pl.ds(64 + o, 16)]
                               + a3 * rows_buf[e, pl.ds(96 + o, 16)])
                zero = jnp.zeros((16,), _F32)
                for blk in range(4):
                    hit = col == blk * 32
                    msg_v[e, pl.ds(blk * 32, 16)] = jnp.where(hit, acc[0],
                                                              zero)
                    msg_v[e, pl.ds(blk * 32 + 16, 16)] = jnp.where(hit, acc[1],
                                                                   zero)
            return _
        lax.fori_loop(0, CHUNK // 16, _group, None)
        pltpu.sync_copy(msg_v, accum.at[drow_v.at[j]], add=True)

    _wait(_gather(0, rows0, sem0, ea0, sea0))

    def _pair(t, _):
        j0 = 2 * t
        h1 = _gather(j0 + 1, rows1, sem1, ea1, sea1)
        _combine_scatter(j0, rows0, ea0)
        _wait(h1)
        j2 = jnp.where(j0 + 2 < NCW, j0 + 2, 0)
        h0 = _gather(j2, rows0, sem0, ea0, sea0)
        _combine_scatter(j0 + 1, rows1, ea1)
        _wait(h0)
        return _
    lax.fori_loop(0, NCW // 2, _pair, None)

    plsc.subcore_barrier()
    pltpu.sync_copy(accum.at[pl.ds(sid * RPT, RPT)],
                    out_hbm.at[pl.ds(cid * AROWS + sid * RPT, RPT)])


def _sc_edge(y, srcf, drow2, dcolf, eaf):
    mesh = plsc.VectorSubcoreMesh(core_axis_name="c", subcore_axis_name="s",
                                  num_cores=2, num_subcores=NS)
    fn = pl.kernel(
        _sc_edge_body,
        out_type=jax.ShapeDtypeStruct((2 * AROWS, 128), _F32),
        mesh=mesh,
        scratch_types=[
            pltpu.VMEM((NCW * CHUNK,), jnp.int32),
            pltpu.VMEM((NCW * CHUNK,), jnp.int32),
            pltpu.VMEM((NCW, CHUNK), jnp.int32),
            pltpu.VMEM((CHUNK, 128), _F32),
            pltpu.VMEM((CHUNK, 128), _F32),
            pltpu.VMEM((CHUNK, 128), _F32),
            pltpu.VMEM_SHARED((AROWS, 128), _F32),
            pltpu.SemaphoreType.DMA,
            pltpu.SemaphoreType.DMA,
            pltpu.SemaphoreType.DMA,
            pltpu.SemaphoreType.DMA,
            pltpu.VMEM((CHUNK * NE,), _F32),
            pltpu.VMEM((CHUNK * NE,), _F32),
        ],
    )
    return fn(y, srcf, drow2, dcolf, eaf)


def _front_body(x_ref, wc_ref, w11_ref, b11_ref, w12_ref, b12_ref,
                y_ref, g_ref):
    x = x_ref[...]
    y_ref[...] = jnp.dot(x, wc_ref[...], preferred_element_type=_F32,
                         precision=_HIGH)
    g_ref[...] = (jnp.tanh(_dotb(x, w11_ref[...]) + b11_ref[...])
                  * jnp.tanh(_dotb(x, w12_ref[...]) + b12_ref[...]))


def _tc_front(x, wc, w11, b11, w12, b12):
    nin = x.shape[1]
    return pl.pallas_call(
        _front_body,
        out_shape=[jax.ShapeDtypeStruct((N, 128), _F32),
                   jax.ShapeDtypeStruct((N, NOUT2), _F32)],
    )(x, wc, w11, b11, w12, b12)


def _bn_block(parts, g_prev, bconv, bng, bnb):
    s = parts[0] + parts[1]
    c = jnp.maximum(s[:, :NOUT1] + bconv, 0.0)
    h = jnp.concatenate([c, g_prev], axis=1)
    mu = jnp.mean(h, axis=0, keepdims=True)
    var = jnp.mean((h - mu) ** 2, axis=0, keepdims=True)
    return bng * (h - mu) * lax.rsqrt(var + 1e-5) + bnb


def _mid_body(part_ref, g_ref, bconv_ref, bng_ref, bnb_ref,
              wc_ref, w11_ref, b11_ref, w12_ref, b12_ref,
              y_ref, gout_ref):
    hn = _bn_block(part_ref[...], g_ref[...], bconv_ref[...],
                   bng_ref[...], bnb_ref[...])
    y_ref[...] = jnp.dot(hn, wc_ref[...], preferred_element_type=_F32,
                         precision=_HIGH)
    gout_ref[...] = (jnp.tanh(_dotb(hn, w11_ref[...]) + b11_ref[...])
                     * jnp.tanh(_dotb(hn, w12_ref[...]) + b12_ref[...]))


def _tc_mid(parts, g_prev, bconv, bng, bnb, wc, w11, b11, w12, b12):
    return pl.pallas_call(
        _mid_body,
        out_shape=[jax.ShapeDtypeStruct((N, 128), _F32),
                   jax.ShapeDtypeStruct((N, NOUT2), _F32)],
    )(parts, g_prev, bconv, bng, bnb, wc, w11, b11, w12, b12)


def _head_body(part_ref, g_ref, bconv_ref, bng_ref, bnb_ref, batch_ref,
               fc1w_ref, fc1b_ref, fc2w_ref, fc2b_ref, out_ref):
    hn = _bn_block(part_ref[...], g_ref[...], bconv_ref[...],
                   bng_ref[...], bnb_ref[...])
    b = batch_ref[...]
    iot = lax.broadcasted_iota(jnp.int32, (NUM_GRAPHS, N), 0)
    oht = (iot == b).astype(_F32)
    s64 = jnp.dot(oht, hn, preferred_element_type=_F32, precision=_HIGH)
    cnt = jnp.sum(oht, axis=1, keepdims=True)
    hg = s64 / jnp.maximum(cnt, 1.0)
    z = jnp.maximum(_dotb(hg, fc1w_ref[...]) + fc1b_ref[...], 0.0)
    out_ref[...] = _dotb(z, fc2w_ref[...]) + fc2b_ref[...]


def _tc_head(parts, g_prev, bconv, bng, bnb, batch2, fc1w, fc1b, fc2w, fc2b):
    return pl.pallas_call(
        _head_body,
        out_shape=jax.ShapeDtypeStruct((NUM_GRAPHS, 1), _F32),
    )(parts, g_prev, bconv, bng, bnb, batch2, fc1w, fc1b, fc2w, fc2b)


def _stack_conv_w(W):
    W = W.astype(_BF16).astype(_F32)
    Wt = jnp.transpose(W, (1, 0, 2))
    Wp = jnp.pad(Wt, ((0, 0), (0, 0), (0, 32 - NOUT1)))
    return Wp.reshape(W.shape[1], NE * 32)


def _pad_worker(a):
    a = a.reshape((NW, EPW) + a.shape[1:])
    pad = [(0, 0), (0, NCW * CHUNK - EPW)] + [(0, 0)] * (a.ndim - 2)
    return jnp.pad(a, pad)


def kernel(x, edge_index2, edge_attr2, batch, params):
    srcf = _pad_worker(edge_index2[0]).reshape(-1)
    dstp = _pad_worker(edge_index2[1])
    drow2 = (dstp // 4).reshape(NW * NCW, CHUNK)
    dcolf = ((dstp % 4) * 32).reshape(-1)
    eaf = _pad_worker(edge_attr2).reshape(-1)
    batch2 = batch.reshape(1, N)

    p = params
    row = lambda v: v.reshape(1, -1)

    wc = [_stack_conv_w(p['conv%d_W' % l]) for l in (1, 2, 3)]

    def edge_pass(y):
        out = _sc_edge(y, srcf, drow2, dcolf, eaf)
        return out.reshape(2, AROWS * 4, 32)[:, :N, :]

    y, g = _tc_front(x, wc[0], p['fc11_1_W'], row(p['fc11_1_b']),
                     p['fc12_1_W'], row(p['fc12_1_b']))
    for l in (2, 3):
        parts = edge_pass(y)
        y, g = _tc_mid(parts, g, row(p['conv%d_b' % (l - 1)]),
                       row(p['bn%d_g' % (l - 1)]), row(p['bn%d_b' % (l - 1)]),
                       wc[l - 1], p['fc11_%d_W' % l], row(p['fc11_%d_b' % l]),
                       p['fc12_%d_W' % l], row(p['fc12_%d_b' % l]))
    parts = edge_pass(y)
    return _tc_head(parts, g, row(p['conv3_b']), row(p['bn3_g']),
                    row(p['bn3_b']), batch2, p['fc1_W'], row(p['fc1_b']),
                    p['fc2_W'], row(p['fc2_b']))
